# Initial kernel scaffold; baseline (speedup 1.0000x reference)
import jax
import jax.numpy as jnp
from jax.experimental import pallas as pl


def kernel(hidden_states, shared_gate, shared_up, shared_down, Wg, Wu, Wd, router_w):
    raise NotImplementedError("write your pallas kernel here")


# speedup vs baseline: 2.1745x; 2.1745x over previous
import functools
import jax
import jax.numpy as jnp
from jax import lax
from jax.experimental import pallas as pl
from jax.experimental.pallas import tpu as pltpu

T = 4096
D = 1024
H = 512
E = 16
K = 2
BM = 256
NT = T * K // BM + E
P_PAD = NT * BM
BT = 512

INTERPRET = False


def _router_body(x_ref, rw_ref, sel_ref, w_ref):
    x = x_ref[...]
    logits = jnp.dot(x, rw_ref[...], preferred_element_type=jnp.float32)
    m = jnp.max(logits, axis=-1, keepdims=True)
    p = jnp.exp(logits - m)
    probs = p / jnp.sum(p, axis=-1, keepdims=True)
    iota = lax.broadcasted_iota(jnp.int32, probs.shape, 1)
    m1 = jnp.max(probs, axis=-1, keepdims=True)
    a1 = jnp.min(jnp.where(probs == m1, iota, E), axis=-1, keepdims=True)
    probs2 = jnp.where(iota == a1, -1.0, probs)
    m2 = jnp.max(probs2, axis=-1, keepdims=True)
    a2 = jnp.min(jnp.where(probs2 == m2, iota, E), axis=-1, keepdims=True)
    ws = m1 + m2
    sel_ref[:, 0:1] = a1
    sel_ref[:, 1:2] = a2
    w_ref[:, 0:1] = m1 / ws
    w_ref[:, 1:2] = m2 / ws


def _router(x, router_w):
    return pl.pallas_call(
        _router_body,
        grid=(T // BT,),
        in_specs=[
            pl.BlockSpec((BT, D), lambda i: (i, 0)),
            pl.BlockSpec((D, E), lambda i: (0, 0)),
        ],
        out_specs=[
            pl.BlockSpec((BT, K), lambda i: (i, 0)),
            pl.BlockSpec((BT, K), lambda i: (i, 0)),
        ],
        out_shape=[
            jax.ShapeDtypeStruct((T, K), jnp.int32),
            jax.ShapeDtypeStruct((T, K), jnp.float32),
        ],
        interpret=INTERPRET,
    )(x, router_w)


NPT = T * K // BM


def _rank_body(e_ref, dest_ref, meta_ref, cnt_ref, off_ref):
    p = pl.program_id(0)
    i = pl.program_id(1)

    @pl.when(i == 0)
    def _():
        cnt_ref[...] = jnp.zeros_like(cnt_ref)

    e = e_ref[0, 0, :].reshape(BM, 1)
    one_hot = (e == lax.broadcasted_iota(jnp.int32, (BM, E), 1)).astype(jnp.float32)
    r = lax.broadcasted_iota(jnp.int32, (BM, BM), 0)
    c = lax.broadcasted_iota(jnp.int32, (BM, BM), 1)
    tri = (c < r).astype(jnp.float32)
    excl = jnp.dot(tri, one_hot, preferred_element_type=jnp.float32)
    carry = cnt_ref[...]
    cnt_ref[...] = carry + jnp.sum(one_hot, axis=0, keepdims=True)

    @pl.when(p == 1)
    def _():
        rank = jnp.sum((excl + carry) * one_hot, axis=1)
        base = jnp.sum(off_ref[...] * one_hot, axis=1)
        dest_ref[0, 0, :] = (rank + base).astype(jnp.int32)

    @pl.when((p == 0) & (i == NPT - 1))
    def _():
        total = cnt_ref[...]
        tiles = jnp.ceil(total / BM)
        rr = lax.broadcasted_iota(jnp.int32, (E, E), 0)
        cc = lax.broadcasted_iota(jnp.int32, (E, E), 1)
        tri_e = (rr < cc).astype(jnp.float32)
        tile_start = jnp.dot(tiles, tri_e, preferred_element_type=jnp.float32)
        off_ref[...] = tile_start * BM
        used = jnp.sum(tiles)
        j2 = lax.broadcasted_iota(jnp.int32, (128, E), 0).astype(jnp.float32)
        ts_b = jnp.broadcast_to(tile_start, (128, E))
        te = jnp.sum((ts_b <= j2).astype(jnp.float32), axis=1) - 1.0
        jrow = lax.broadcasted_iota(jnp.int32, (1, 128), 1).astype(jnp.float32)
        meta_ref[0:1, :] = te.reshape(1, 128).astype(jnp.int32)
        meta_ref[1:2, :] = (jrow < used).astype(jnp.int32)


def _rank(e_pairs3):
    return pl.pallas_call(
        _rank_body,
        grid=(2, NPT),
        in_specs=[pl.BlockSpec((1, 1, BM), lambda p, i: (i, 0, 0))],
        out_specs=[
            pl.BlockSpec((1, 1, BM), lambda p, i: (i, 0, 0)),
            pl.BlockSpec((8, 128), lambda p, i: (0, 0)),
        ],
        out_shape=[
            jax.ShapeDtypeStruct((NPT, 1, BM), jnp.int32),
            jax.ShapeDtypeStruct((8, 128), jnp.int32),
        ],
        scratch_shapes=[
            pltpu.VMEM((1, E), jnp.float32),
            pltpu.VMEM((1, E), jnp.float32),
        ],
        interpret=INTERPRET,
    )(e_pairs3)


def _ffn_body(te_ref, valid_ref, xs_ref, wg_ref, wu_ref, wd_ref, ys_ref):
    i = pl.program_id(0)

    @pl.when(valid_ref[i] == 1)
    def _():
        x = xs_ref[...]
        g = jnp.dot(x, wg_ref[0], preferred_element_type=jnp.float32)
        u = jnp.dot(x, wu_ref[0], preferred_element_type=jnp.float32)
        h = g * (1.0 / (1.0 + jnp.exp(-g))) * u
        ys_ref[...] = jnp.dot(h, wd_ref[0], preferred_element_type=jnp.float32)


def _ffn(xs, Wg, Wu, Wd, te, valid):
    grid_spec = pltpu.PrefetchScalarGridSpec(
        num_scalar_prefetch=2,
        grid=(NT,),
        in_specs=[
            pl.BlockSpec((BM, D), lambda i, te, va: (i, 0)),
            pl.BlockSpec((1, D, H), lambda i, te, va: (te[i], 0, 0)),
            pl.BlockSpec((1, D, H), lambda i, te, va: (te[i], 0, 0)),
            pl.BlockSpec((1, H, D), lambda i, te, va: (te[i], 0, 0)),
        ],
        out_specs=pl.BlockSpec((BM, D), lambda i, te, va: (i, 0)),
    )
    return pl.pallas_call(
        _ffn_body,
        grid_spec=grid_spec,
        out_shape=jax.ShapeDtypeStruct((P_PAD, D), jnp.float32),
        interpret=INTERPRET,
    )(te, valid, xs, Wg, Wu, Wd)


def _shared_body(x_ref, sg_ref, su_ref, sd_ref, y0_ref, y1_ref, w0_ref, w1_ref, o_ref):
    x = x_ref[...]
    g = jnp.dot(x, sg_ref[...], preferred_element_type=jnp.float32)
    u = jnp.dot(x, su_ref[...], preferred_element_type=jnp.float32)
    h = g * (1.0 / (1.0 + jnp.exp(-g))) * u
    s = jnp.dot(h, sd_ref[...], preferred_element_type=jnp.float32)
    w0 = w0_ref[0].reshape(BT, 1)
    w1 = w1_ref[0].reshape(BT, 1)
    o_ref[...] = s + w0 * y0_ref[0] + w1 * y1_ref[0]


def _shared_combine(x, sg, su, sd, y_pairs, w3):
    return pl.pallas_call(
        _shared_body,
        grid=(T // BT,),
        in_specs=[
            pl.BlockSpec((BT, D), lambda i: (i, 0)),
            pl.BlockSpec((D, H), lambda i: (0, 0)),
            pl.BlockSpec((D, H), lambda i: (0, 0)),
            pl.BlockSpec((H, D), lambda i: (0, 0)),
            pl.BlockSpec((1, BT, D), lambda i: (0, i, 0)),
            pl.BlockSpec((1, BT, D), lambda i: (1, i, 0)),
            pl.BlockSpec((1, 1, BT), lambda i: (i, 0, 0)),
            pl.BlockSpec((1, 1, BT), lambda i: (i + T // BT, 0, 0)),
        ],
        out_specs=pl.BlockSpec((BT, D), lambda i: (i, 0)),
        out_shape=jax.ShapeDtypeStruct((T, D), jnp.float32),
        interpret=INTERPRET,
    )(x, sg, su, sd, y_pairs, y_pairs, w3, w3)


def _dispatch(x, dest_k):
    xs = jnp.zeros((P_PAD, D), jnp.float32)
    d = dest_k.reshape(K, T)
    xs = xs.at[d[0]].set(x)
    xs = xs.at[d[1]].set(x)
    return xs


def _combine_gather(ys, dest_k):
    d = dest_k.reshape(K, T)
    return ys[d.reshape(-1)].reshape(K, T, D)


def kernel(hidden_states, shared_gate, shared_up, shared_down, Wg, Wu, Wd, router_w):
    b, l, d = hidden_states.shape
    x = hidden_states.reshape(T, D)

    sel, w = _router(x, router_w)

    e_pairs3 = sel.reshape(NPT, 1, BM)
    dest3, meta = _rank(e_pairs3)
    te = meta[0, :NT]
    valid = meta[1, :NT]

    dest_k = dest3.reshape(T, K).T

    xs = _dispatch(x, dest_k)
    ys = _ffn(xs, Wg, Wu, Wd, te, valid)
    y_pairs = _combine_gather(ys, dest_k)

    w3 = w.T.reshape(K * T // BT, 1, BT)
    out = _shared_combine(x, shared_gate, shared_up, shared_down, y_pairs, w3)
    return out.reshape(b, l, d)


# speedup vs baseline: 2.6036x; 1.1974x over previous
import functools
import jax
import jax.numpy as jnp
from jax import lax
from jax.experimental import pallas as pl
from jax.experimental.pallas import tpu as pltpu
from jax.experimental.pallas import tpu_sc as plsc

T = 4096
D = 1024
H = 512
E = 16
K = 2
BM = 256
NT = T * K // BM + E
P_PAD = NT * BM
BT = 512

INTERPRET = False


def _router_body(x_ref, rw_ref, sel_ref, w_ref):
    x = x_ref[...]
    logits = jnp.dot(x, rw_ref[...], preferred_element_type=jnp.float32)
    m = jnp.max(logits, axis=-1, keepdims=True)
    p = jnp.exp(logits - m)
    probs = p / jnp.sum(p, axis=-1, keepdims=True)
    iota = lax.broadcasted_iota(jnp.int32, probs.shape, 1)
    m1 = jnp.max(probs, axis=-1, keepdims=True)
    a1 = jnp.min(jnp.where(probs == m1, iota, E), axis=-1, keepdims=True)
    probs2 = jnp.where(iota == a1, -1.0, probs)
    m2 = jnp.max(probs2, axis=-1, keepdims=True)
    a2 = jnp.min(jnp.where(probs2 == m2, iota, E), axis=-1, keepdims=True)
    ws = m1 + m2
    sel_ref[:, 0:1] = a1
    sel_ref[:, 1:2] = a2
    w_ref[:, 0:1] = m1 / ws
    w_ref[:, 1:2] = m2 / ws


def _router(x, router_w):
    return pl.pallas_call(
        _router_body,
        grid=(T // BT,),
        in_specs=[
            pl.BlockSpec((BT, D), lambda i: (i, 0)),
            pl.BlockSpec((D, E), lambda i: (0, 0)),
        ],
        out_specs=[
            pl.BlockSpec((BT, K), lambda i: (i, 0)),
            pl.BlockSpec((BT, K), lambda i: (i, 0)),
        ],
        out_shape=[
            jax.ShapeDtypeStruct((T, K), jnp.int32),
            jax.ShapeDtypeStruct((T, K), jnp.float32),
        ],
        interpret=INTERPRET,
    )(x, router_w)


NPT = T * K // BM


def _rank_body(e_ref, dest_ref, meta_ref, cnt_ref, off_ref):
    p = pl.program_id(0)
    i = pl.program_id(1)

    @pl.when(i == 0)
    def _():
        cnt_ref[...] = jnp.zeros_like(cnt_ref)

    e = e_ref[0, 0, :].reshape(BM, 1)
    one_hot = (e == lax.broadcasted_iota(jnp.int32, (BM, E), 1)).astype(jnp.float32)
    r = lax.broadcasted_iota(jnp.int32, (BM, BM), 0)
    c = lax.broadcasted_iota(jnp.int32, (BM, BM), 1)
    tri = (c < r).astype(jnp.float32)
    excl = jnp.dot(tri, one_hot, preferred_element_type=jnp.float32)
    carry = cnt_ref[...]
    cnt_ref[...] = carry + jnp.sum(one_hot, axis=0, keepdims=True)

    @pl.when(p == 1)
    def _():
        rank = jnp.sum((excl + carry) * one_hot, axis=1)
        base = jnp.sum(off_ref[...] * one_hot, axis=1)
        dest_ref[0, 0, :] = (rank + base).astype(jnp.int32)

    @pl.when((p == 0) & (i == NPT - 1))
    def _():
        total = cnt_ref[...]
        tiles = jnp.ceil(total / BM)
        rr = lax.broadcasted_iota(jnp.int32, (E, E), 0)
        cc = lax.broadcasted_iota(jnp.int32, (E, E), 1)
        tri_e = (rr < cc).astype(jnp.float32)
        tile_start = jnp.dot(tiles, tri_e, preferred_element_type=jnp.float32)
        off_ref[...] = tile_start * BM
        used = jnp.sum(tiles)
        j2 = lax.broadcasted_iota(jnp.int32, (128, E), 0).astype(jnp.float32)
        ts_b = jnp.broadcast_to(tile_start, (128, E))
        te = jnp.sum((ts_b <= j2).astype(jnp.float32), axis=1) - 1.0
        jrow = lax.broadcasted_iota(jnp.int32, (1, 128), 1).astype(jnp.float32)
        meta_ref[0:1, :] = te.reshape(1, 128).astype(jnp.int32)
        meta_ref[1:2, :] = (jrow < used).astype(jnp.int32)


def _rank(e_pairs3):
    return pl.pallas_call(
        _rank_body,
        grid=(2, NPT),
        in_specs=[pl.BlockSpec((1, 1, BM), lambda p, i: (i, 0, 0))],
        out_specs=[
            pl.BlockSpec((1, 1, BM), lambda p, i: (i, 0, 0)),
            pl.BlockSpec((8, 128), lambda p, i: (0, 0)),
        ],
        out_shape=[
            jax.ShapeDtypeStruct((NPT, 1, BM), jnp.int32),
            jax.ShapeDtypeStruct((8, 128), jnp.int32),
        ],
        scratch_shapes=[
            pltpu.VMEM((1, E), jnp.float32),
            pltpu.VMEM((1, E), jnp.float32),
        ],
        interpret=INTERPRET,
    )(e_pairs3)


def _ffn_body(te_ref, valid_ref, xs_ref, wg_ref, wu_ref, wd_ref, ys_ref):
    i = pl.program_id(0)

    @pl.when(valid_ref[i] == 1)
    def _():
        x = xs_ref[...]
        g = jnp.dot(x, wg_ref[0], preferred_element_type=jnp.float32)
        u = jnp.dot(x, wu_ref[0], preferred_element_type=jnp.float32)
        h = g * (1.0 / (1.0 + jnp.exp(-g))) * u
        ys_ref[...] = jnp.dot(h, wd_ref[0], preferred_element_type=jnp.float32)


def _ffn(xs, Wg, Wu, Wd, te, valid):
    grid_spec = pltpu.PrefetchScalarGridSpec(
        num_scalar_prefetch=2,
        grid=(NT,),
        in_specs=[
            pl.BlockSpec((BM, D), lambda i, te, va: (i, 0)),
            pl.BlockSpec((1, D, H), lambda i, te, va: (te[i], 0, 0)),
            pl.BlockSpec((1, D, H), lambda i, te, va: (te[i], 0, 0)),
            pl.BlockSpec((1, H, D), lambda i, te, va: (te[i], 0, 0)),
        ],
        out_specs=pl.BlockSpec((BM, D), lambda i, te, va: (i, 0)),
    )
    return pl.pallas_call(
        _ffn_body,
        grid_spec=grid_spec,
        out_shape=jax.ShapeDtypeStruct((P_PAD, D), jnp.float32),
        interpret=INTERPRET,
    )(te, valid, xs, Wg, Wu, Wd)


def _shared_body(x_ref, sg_ref, su_ref, sd_ref, y0_ref, y1_ref, w0_ref, w1_ref, o_ref):
    x = x_ref[...]
    g = jnp.dot(x, sg_ref[...], preferred_element_type=jnp.float32)
    u = jnp.dot(x, su_ref[...], preferred_element_type=jnp.float32)
    h = g * (1.0 / (1.0 + jnp.exp(-g))) * u
    s = jnp.dot(h, sd_ref[...], preferred_element_type=jnp.float32)
    w0 = w0_ref[0].reshape(BT, 1)
    w1 = w1_ref[0].reshape(BT, 1)
    o_ref[...] = s + w0 * y0_ref[0] + w1 * y1_ref[0]


def _shared_combine(x, sg, su, sd, y_pairs, w3):
    return pl.pallas_call(
        _shared_body,
        grid=(T // BT,),
        in_specs=[
            pl.BlockSpec((BT, D), lambda i: (i, 0)),
            pl.BlockSpec((D, H), lambda i: (0, 0)),
            pl.BlockSpec((D, H), lambda i: (0, 0)),
            pl.BlockSpec((H, D), lambda i: (0, 0)),
            pl.BlockSpec((1, BT, D), lambda i: (0, i, 0)),
            pl.BlockSpec((1, BT, D), lambda i: (1, i, 0)),
            pl.BlockSpec((1, 1, BT), lambda i: (i, 0, 0)),
            pl.BlockSpec((1, 1, BT), lambda i: (i + T // BT, 0, 0)),
        ],
        out_specs=pl.BlockSpec((BT, D), lambda i: (i, 0)),
        out_shape=jax.ShapeDtypeStruct((T, D), jnp.float32),
        interpret=INTERPRET,
    )(x, sg, su, sd, y_pairs, y_pairs, w3, w3)


NW = 32
TPW = T // NW
NCH = TPW // 16


def _sc_mesh():
    return plsc.VectorSubcoreMesh(core_axis_name="c", subcore_axis_name="s")


def _dispatch(x, dest_k4):
    @functools.partial(
        pl.kernel,
        mesh=_sc_mesh(),
        out_type=jax.ShapeDtypeStruct((P_PAD, D), jnp.float32),
        scratch_types=[
            pltpu.VMEM((NCH, 16), jnp.int32),
            pltpu.VMEM((NCH, 16), jnp.int32),
            pltpu.VMEM((16, D), jnp.float32),
            pltpu.SemaphoreType.DMA,
        ],
    )
    def k(x_hbm, d_hbm, xs_hbm, idx0_v, idx1_v, rows_v, sem):
        w = lax.axis_index("s") * 2 + lax.axis_index("c")
        pltpu.sync_copy(d_hbm.at[0, w], idx0_v)
        pltpu.sync_copy(d_hbm.at[1, w], idx1_v)
        for ch in range(NCH):
            pltpu.sync_copy(x_hbm.at[pl.ds(w * TPW + ch * 16, 16)], rows_v)
            c0 = pltpu.async_copy(rows_v, xs_hbm.at[idx0_v.at[ch]], sem)
            c1 = pltpu.async_copy(rows_v, xs_hbm.at[idx1_v.at[ch]], sem)
            c0.wait()
            c1.wait()

    return k(x, dest_k4)


def _combine_gather(ys, dest_k4):
    @functools.partial(
        pl.kernel,
        mesh=_sc_mesh(),
        out_type=jax.ShapeDtypeStruct((K, T, D), jnp.float32),
        scratch_types=[
            pltpu.VMEM((NCH, 16), jnp.int32),
            pltpu.VMEM((16, D), jnp.float32),
            pltpu.SemaphoreType.DMA,
        ],
    )
    def k(ys_hbm, d_hbm, yp_hbm, idx_v, rows_v, sem):
        w = lax.axis_index("s") * 2 + lax.axis_index("c")
        for kk in range(K):
            pltpu.sync_copy(d_hbm.at[kk, w], idx_v)
            for ch in range(NCH):
                pltpu.async_copy(ys_hbm.at[idx_v.at[ch]], rows_v, sem).wait()
                pltpu.sync_copy(rows_v, yp_hbm.at[kk, pl.ds(w * TPW + ch * 16, 16)])

    return k(ys, dest_k4)


def kernel(hidden_states, shared_gate, shared_up, shared_down, Wg, Wu, Wd, router_w):
    b, l, d = hidden_states.shape
    x = hidden_states.reshape(T, D)

    sel, w = _router(x, router_w)

    e_pairs3 = sel.reshape(NPT, 1, BM)
    dest3, meta = _rank(e_pairs3)
    te = meta[0, :NT]
    valid = meta[1, :NT]

    dest_k4 = dest3.reshape(T, K).T.reshape(K, NW, NCH, 16)

    xs = _dispatch(x, dest_k4)
    ys = _ffn(xs, Wg, Wu, Wd, te, valid)
    y_pairs = _combine_gather(ys, dest_k4)

    w3 = w.T.reshape(K * T // BT, 1, BT)
    out = _shared_combine(x, shared_gate, shared_up, shared_down, y_pairs, w3)
    return out.reshape(b, l, d)


# speedup vs baseline: 2.6042x; 1.0002x over previous
import functools
import jax
import jax.numpy as jnp
from jax import lax
from jax.experimental import pallas as pl
from jax.experimental.pallas import tpu as pltpu
from jax.experimental.pallas import tpu_sc as plsc

T = 4096
D = 1024
H = 512
E = 16
K = 2
BM = 256
NT = T * K // BM + E
P_PAD = NT * BM
BT = 512

INTERPRET = False


def _router_body(x_ref, rw_ref, sel_ref, w_ref):
    x = x_ref[...]
    logits = jnp.dot(x, rw_ref[...], preferred_element_type=jnp.float32)
    m = jnp.max(logits, axis=-1, keepdims=True)
    p = jnp.exp(logits - m)
    probs = p / jnp.sum(p, axis=-1, keepdims=True)
    iota = lax.broadcasted_iota(jnp.int32, probs.shape, 1)
    m1 = jnp.max(probs, axis=-1, keepdims=True)
    a1 = jnp.min(jnp.where(probs == m1, iota, E), axis=-1, keepdims=True)
    probs2 = jnp.where(iota == a1, -1.0, probs)
    m2 = jnp.max(probs2, axis=-1, keepdims=True)
    a2 = jnp.min(jnp.where(probs2 == m2, iota, E), axis=-1, keepdims=True)
    ws = m1 + m2
    sel_ref[:, 0:1] = a1
    sel_ref[:, 1:2] = a2
    w_ref[:, 0:1] = m1 / ws
    w_ref[:, 1:2] = m2 / ws


def _router(x, router_w):
    return pl.pallas_call(
        _router_body,
        grid=(T // BT,),
        in_specs=[
            pl.BlockSpec((BT, D), lambda i: (i, 0)),
            pl.BlockSpec((D, E), lambda i: (0, 0)),
        ],
        out_specs=[
            pl.BlockSpec((BT, K), lambda i: (i, 0)),
            pl.BlockSpec((BT, K), lambda i: (i, 0)),
        ],
        out_shape=[
            jax.ShapeDtypeStruct((T, K), jnp.int32),
            jax.ShapeDtypeStruct((T, K), jnp.float32),
        ],
        interpret=INTERPRET,
    )(x, router_w)


NPT = T * K // BM


def _rank_body(e_ref, dest_ref, meta_ref, cnt_ref, off_ref):
    p = pl.program_id(0)
    i = pl.program_id(1)

    @pl.when(i == 0)
    def _():
        cnt_ref[...] = jnp.zeros_like(cnt_ref)

    e = e_ref[0, 0, :].reshape(BM, 1)
    one_hot = (e == lax.broadcasted_iota(jnp.int32, (BM, E), 1)).astype(jnp.float32)
    r = lax.broadcasted_iota(jnp.int32, (BM, BM), 0)
    c = lax.broadcasted_iota(jnp.int32, (BM, BM), 1)
    tri = (c < r).astype(jnp.float32)
    excl = jnp.dot(tri, one_hot, preferred_element_type=jnp.float32)
    carry = cnt_ref[...]
    cnt_ref[...] = carry + jnp.sum(one_hot, axis=0, keepdims=True)

    @pl.when(p == 1)
    def _():
        rank = jnp.sum((excl + carry) * one_hot, axis=1)
        base = jnp.sum(off_ref[...] * one_hot, axis=1)
        dest_ref[0, 0, :] = (rank + base).astype(jnp.int32)

    @pl.when((p == 0) & (i == NPT - 1))
    def _():
        total = cnt_ref[...]
        tiles = jnp.ceil(total / BM)
        rr = lax.broadcasted_iota(jnp.int32, (E, E), 0)
        cc = lax.broadcasted_iota(jnp.int32, (E, E), 1)
        tri_e = (rr < cc).astype(jnp.float32)
        tile_start = jnp.dot(tiles, tri_e, preferred_element_type=jnp.float32)
        off_ref[...] = tile_start * BM
        used = jnp.sum(tiles)
        j2 = lax.broadcasted_iota(jnp.int32, (128, E), 0).astype(jnp.float32)
        ts_b = jnp.broadcast_to(tile_start, (128, E))
        te = jnp.sum((ts_b <= j2).astype(jnp.float32), axis=1) - 1.0
        jrow = lax.broadcasted_iota(jnp.int32, (1, 128), 1).astype(jnp.float32)
        meta_ref[0:1, :] = te.reshape(1, 128).astype(jnp.int32)
        meta_ref[1:2, :] = (jrow < used).astype(jnp.int32)


def _rank(e_pairs3):
    return pl.pallas_call(
        _rank_body,
        grid=(2, NPT),
        in_specs=[pl.BlockSpec((1, 1, BM), lambda p, i: (i, 0, 0))],
        out_specs=[
            pl.BlockSpec((1, 1, BM), lambda p, i: (i, 0, 0)),
            pl.BlockSpec((8, 128), lambda p, i: (0, 0)),
        ],
        out_shape=[
            jax.ShapeDtypeStruct((NPT, 1, BM), jnp.int32),
            jax.ShapeDtypeStruct((8, 128), jnp.int32),
        ],
        scratch_shapes=[
            pltpu.VMEM((1, E), jnp.float32),
            pltpu.VMEM((1, E), jnp.float32),
        ],
        interpret=INTERPRET,
    )(e_pairs3)


def _ffn_body(te_ref, valid_ref, xs_ref, wg_ref, wu_ref, wd_ref, ys_ref):
    i = pl.program_id(0)

    @pl.when(valid_ref[i] == 1)
    def _():
        x = xs_ref[...].astype(jnp.bfloat16)
        g = jnp.dot(x, wg_ref[0].astype(jnp.bfloat16), preferred_element_type=jnp.float32)
        u = jnp.dot(x, wu_ref[0].astype(jnp.bfloat16), preferred_element_type=jnp.float32)
        h = g * (1.0 / (1.0 + jnp.exp(-g))) * u
        ys_ref[...] = jnp.dot(h.astype(jnp.bfloat16), wd_ref[0].astype(jnp.bfloat16),
                              preferred_element_type=jnp.float32)


def _ffn(xs, Wg, Wu, Wd, te, valid):
    grid_spec = pltpu.PrefetchScalarGridSpec(
        num_scalar_prefetch=2,
        grid=(NT,),
        in_specs=[
            pl.BlockSpec((BM, D), lambda i, te, va: (i, 0)),
            pl.BlockSpec((1, D, H), lambda i, te, va: (te[i], 0, 0)),
            pl.BlockSpec((1, D, H), lambda i, te, va: (te[i], 0, 0)),
            pl.BlockSpec((1, H, D), lambda i, te, va: (te[i], 0, 0)),
        ],
        out_specs=pl.BlockSpec((BM, D), lambda i, te, va: (i, 0)),
    )
    return pl.pallas_call(
        _ffn_body,
        grid_spec=grid_spec,
        out_shape=jax.ShapeDtypeStruct((P_PAD, D), jnp.float32),
        interpret=INTERPRET,
    )(te, valid, xs, Wg, Wu, Wd)


def _shared_body(x_ref, sg_ref, su_ref, sd_ref, y0_ref, y1_ref, w0_ref, w1_ref, o_ref):
    x = x_ref[...].astype(jnp.bfloat16)
    g = jnp.dot(x, sg_ref[...].astype(jnp.bfloat16), preferred_element_type=jnp.float32)
    u = jnp.dot(x, su_ref[...].astype(jnp.bfloat16), preferred_element_type=jnp.float32)
    h = g * (1.0 / (1.0 + jnp.exp(-g))) * u
    s = jnp.dot(h.astype(jnp.bfloat16), sd_ref[...].astype(jnp.bfloat16),
                preferred_element_type=jnp.float32)
    w0 = w0_ref[0].reshape(BT, 1)
    w1 = w1_ref[0].reshape(BT, 1)
    o_ref[...] = s + w0 * y0_ref[0] + w1 * y1_ref[0]


def _shared_combine(x, sg, su, sd, y_pairs, w3):
    return pl.pallas_call(
        _shared_body,
        grid=(T // BT,),
        in_specs=[
            pl.BlockSpec((BT, D), lambda i: (i, 0)),
            pl.BlockSpec((D, H), lambda i: (0, 0)),
            pl.BlockSpec((D, H), lambda i: (0, 0)),
            pl.BlockSpec((H, D), lambda i: (0, 0)),
            pl.BlockSpec((1, BT, D), lambda i: (0, i, 0)),
            pl.BlockSpec((1, BT, D), lambda i: (1, i, 0)),
            pl.BlockSpec((1, 1, BT), lambda i: (i, 0, 0)),
            pl.BlockSpec((1, 1, BT), lambda i: (i + T // BT, 0, 0)),
        ],
        out_specs=pl.BlockSpec((BT, D), lambda i: (i, 0)),
        out_shape=jax.ShapeDtypeStruct((T, D), jnp.float32),
        interpret=INTERPRET,
    )(x, sg, su, sd, y_pairs, y_pairs, w3, w3)


NW = 32
TPW = T // NW
NCH = TPW // 16


def _sc_mesh():
    return plsc.VectorSubcoreMesh(core_axis_name="c", subcore_axis_name="s")


def _dispatch(x, dest_k4):
    @functools.partial(
        pl.kernel,
        mesh=_sc_mesh(),
        out_type=jax.ShapeDtypeStruct((P_PAD, D), jnp.float32),
        scratch_types=[
            pltpu.VMEM((NCH, 16), jnp.int32),
            pltpu.VMEM((NCH, 16), jnp.int32),
            pltpu.VMEM((16, D), jnp.float32),
            pltpu.SemaphoreType.DMA,
        ],
    )
    def k(x_hbm, d_hbm, xs_hbm, idx0_v, idx1_v, rows_v, sem):
        w = lax.axis_index("s") * 2 + lax.axis_index("c")
        pltpu.sync_copy(d_hbm.at[0, w], idx0_v)
        pltpu.sync_copy(d_hbm.at[1, w], idx1_v)
        for ch in range(NCH):
            pltpu.sync_copy(x_hbm.at[pl.ds(w * TPW + ch * 16, 16)], rows_v)
            c0 = pltpu.async_copy(rows_v, xs_hbm.at[idx0_v.at[ch]], sem)
            c1 = pltpu.async_copy(rows_v, xs_hbm.at[idx1_v.at[ch]], sem)
            c0.wait()
            c1.wait()

    return k(x, dest_k4)


def _combine_gather(ys, dest_k4):
    @functools.partial(
        pl.kernel,
        mesh=_sc_mesh(),
        out_type=jax.ShapeDtypeStruct((K, T, D), jnp.float32),
        scratch_types=[
            pltpu.VMEM((NCH, 16), jnp.int32),
            pltpu.VMEM((16, D), jnp.float32),
            pltpu.SemaphoreType.DMA,
        ],
    )
    def k(ys_hbm, d_hbm, yp_hbm, idx_v, rows_v, sem):
        w = lax.axis_index("s") * 2 + lax.axis_index("c")
        for kk in range(K):
            pltpu.sync_copy(d_hbm.at[kk, w], idx_v)
            for ch in range(NCH):
                pltpu.async_copy(ys_hbm.at[idx_v.at[ch]], rows_v, sem).wait()
                pltpu.sync_copy(rows_v, yp_hbm.at[kk, pl.ds(w * TPW + ch * 16, 16)])

    return k(ys, dest_k4)


def kernel(hidden_states, shared_gate, shared_up, shared_down, Wg, Wu, Wd, router_w):
    b, l, d = hidden_states.shape
    x = hidden_states.reshape(T, D)

    sel, w = _router(x, router_w)

    e_pairs3 = sel.reshape(NPT, 1, BM)
    dest3, meta = _rank(e_pairs3)
    te = meta[0, :NT]
    valid = meta[1, :NT]

    dest_k4 = dest3.reshape(T, K).T.reshape(K, NW, NCH, 16)

    xs = _dispatch(x, dest_k4)
    ys = _ffn(xs, Wg, Wu, Wd, te, valid)
    y_pairs = _combine_gather(ys, dest_k4)

    w3 = w.T.reshape(K * T // BT, 1, BT)
    out = _shared_combine(x, shared_gate, shared_up, shared_down, y_pairs, w3)
    return out.reshape(b, l, d)


# speedup vs baseline: 2.8245x; 1.0846x over previous
import functools
import jax
import jax.numpy as jnp
from jax import lax
from jax.experimental import pallas as pl
from jax.experimental.pallas import tpu as pltpu
from jax.experimental.pallas import tpu_sc as plsc

T = 4096
D = 1024
H = 512
E = 16
K = 2
BM = 256
NT = T * K // BM + E
P_PAD = NT * BM
BT = 512

INTERPRET = False


NRT = T // BT


def _top2(x, rw):
    logits = jnp.dot(x, rw, preferred_element_type=jnp.float32)
    m = jnp.max(logits, axis=-1, keepdims=True)
    pe = jnp.exp(logits - m)
    probs = pe / jnp.sum(pe, axis=-1, keepdims=True)
    iota = lax.broadcasted_iota(jnp.int32, probs.shape, 1)
    m1 = jnp.max(probs, axis=-1, keepdims=True)
    a1 = jnp.min(jnp.where(probs == m1, iota, E), axis=-1, keepdims=True)
    probs2 = jnp.where(iota == a1, -1.0, probs)
    m2 = jnp.max(probs2, axis=-1, keepdims=True)
    a2 = jnp.min(jnp.where(probs2 == m2, iota, E), axis=-1, keepdims=True)
    ws = m1 + m2
    return a1, a2, m1 / ws, m2 / ws


def _route_rank_body(x_ref, rw_ref, d0_ref, d1_ref, w0_ref, w1_ref, meta_ref,
                     cnt_ref, off_ref):
    p = pl.program_id(0)
    i = pl.program_id(1)

    @pl.when(i == 0)
    def _():
        cnt_ref[...] = jnp.zeros_like(cnt_ref)

    a1, a2, wn1, wn2 = _top2(x_ref[...], rw_ref[...])
    ie = lax.broadcasted_iota(jnp.int32, (BT, E), 1)
    oh0 = (a1 == ie).astype(jnp.float32)
    oh1 = (a2 == ie).astype(jnp.float32)
    both = oh0 + oh1
    carry = cnt_ref[...]
    cnt_ref[...] = carry + jnp.sum(both, axis=0, keepdims=True)

    @pl.when((p == 0) & (i == NRT - 1))
    def _():
        total = cnt_ref[...]
        tiles = jnp.ceil(total / BM)
        rr = lax.broadcasted_iota(jnp.int32, (E, E), 0)
        cc = lax.broadcasted_iota(jnp.int32, (E, E), 1)
        tri_e = (rr < cc).astype(jnp.float32)
        off_ref[...] = jnp.dot(tiles, tri_e, preferred_element_type=jnp.float32) * BM

    @pl.when(p == 1)
    def _():
        r = lax.broadcasted_iota(jnp.int32, (BT, BT), 0)
        c = lax.broadcasted_iota(jnp.int32, (BT, BT), 1)
        tri = (c < r).astype(jnp.float32)
        excl = jnp.dot(tri, both, preferred_element_type=jnp.float32)
        base = excl + carry + off_ref[...]
        d0_ref[0, 0, :] = jnp.sum(base * oh0, axis=1).astype(jnp.int32)
        d1_ref[0, 0, :] = jnp.sum((base + oh0) * oh1, axis=1).astype(jnp.int32)
        w0_ref[0, 0, :] = wn1[:, 0]
        w1_ref[0, 0, :] = wn2[:, 0]

        @pl.when(i == NRT - 1)
        def _():
            tile_start = off_ref[...] / BM
            used = jnp.sum(jnp.ceil(cnt_ref[...] / BM))
            j2 = lax.broadcasted_iota(jnp.int32, (128, E), 0).astype(jnp.float32)
            ts_b = jnp.broadcast_to(tile_start, (128, E))
            te = jnp.sum((ts_b <= j2).astype(jnp.float32), axis=1) - 1.0
            jrow = lax.broadcasted_iota(jnp.int32, (1, 128), 1).astype(jnp.float32)
            meta_ref[0:1, :] = te.reshape(1, 128).astype(jnp.int32)
            meta_ref[1:2, :] = (jrow < used).astype(jnp.int32)


def _route_rank(x, router_w):
    return pl.pallas_call(
        _route_rank_body,
        grid=(2, NRT),
        in_specs=[
            pl.BlockSpec((BT, D), lambda p, i: (i, 0)),
            pl.BlockSpec((D, E), lambda p, i: (0, 0)),
        ],
        out_specs=[
            pl.BlockSpec((1, 1, BT), lambda p, i: (i, 0, 0)),
            pl.BlockSpec((1, 1, BT), lambda p, i: (i, 0, 0)),
            pl.BlockSpec((1, 1, BT), lambda p, i: (i, 0, 0)),
            pl.BlockSpec((1, 1, BT), lambda p, i: (i, 0, 0)),
            pl.BlockSpec((8, 128), lambda p, i: (0, 0)),
        ],
        out_shape=[
            jax.ShapeDtypeStruct((NRT, 1, BT), jnp.int32),
            jax.ShapeDtypeStruct((NRT, 1, BT), jnp.int32),
            jax.ShapeDtypeStruct((NRT, 1, BT), jnp.float32),
            jax.ShapeDtypeStruct((NRT, 1, BT), jnp.float32),
            jax.ShapeDtypeStruct((8, 128), jnp.int32),
        ],
        scratch_shapes=[
            pltpu.VMEM((1, E), jnp.float32),
            pltpu.VMEM((1, E), jnp.float32),
        ],
        interpret=INTERPRET,
    )(x, router_w)


def _ffn_body(te_ref, valid_ref, xs_ref, wg_ref, wu_ref, wd_ref, ys_ref):
    i = pl.program_id(0)

    @pl.when(valid_ref[i] == 1)
    def _():
        x = xs_ref[...].astype(jnp.bfloat16)
        g = jnp.dot(x, wg_ref[0].astype(jnp.bfloat16), preferred_element_type=jnp.float32)
        u = jnp.dot(x, wu_ref[0].astype(jnp.bfloat16), preferred_element_type=jnp.float32)
        h = g * (1.0 / (1.0 + jnp.exp(-g))) * u
        ys_ref[...] = jnp.dot(h.astype(jnp.bfloat16), wd_ref[0].astype(jnp.bfloat16),
                              preferred_element_type=jnp.float32)


def _ffn(xs, Wg, Wu, Wd, te, valid):
    grid_spec = pltpu.PrefetchScalarGridSpec(
        num_scalar_prefetch=2,
        grid=(NT,),
        in_specs=[
            pl.BlockSpec((BM, D), lambda i, te, va: (i, 0)),
            pl.BlockSpec((1, D, H), lambda i, te, va: (te[i], 0, 0)),
            pl.BlockSpec((1, D, H), lambda i, te, va: (te[i], 0, 0)),
            pl.BlockSpec((1, H, D), lambda i, te, va: (te[i], 0, 0)),
        ],
        out_specs=pl.BlockSpec((BM, D), lambda i, te, va: (i, 0)),
    )
    return pl.pallas_call(
        _ffn_body,
        grid_spec=grid_spec,
        out_shape=jax.ShapeDtypeStruct((P_PAD, D), jnp.float32),
        interpret=INTERPRET,
    )(te, valid, xs, Wg, Wu, Wd)


def _shared_body(x_ref, sg_ref, su_ref, sd_ref, y0_ref, y1_ref, w0_ref, w1_ref, o_ref):
    x = x_ref[...].astype(jnp.bfloat16)
    g = jnp.dot(x, sg_ref[...].astype(jnp.bfloat16), preferred_element_type=jnp.float32)
    u = jnp.dot(x, su_ref[...].astype(jnp.bfloat16), preferred_element_type=jnp.float32)
    h = g * (1.0 / (1.0 + jnp.exp(-g))) * u
    s = jnp.dot(h.astype(jnp.bfloat16), sd_ref[...].astype(jnp.bfloat16),
                preferred_element_type=jnp.float32)
    w0 = w0_ref[0].reshape(BT, 1)
    w1 = w1_ref[0].reshape(BT, 1)
    o_ref[...] = s + w0 * y0_ref[0] + w1 * y1_ref[0]


def _shared_combine(x, sg, su, sd, y_pairs, w3):
    return pl.pallas_call(
        _shared_body,
        grid=(T // BT,),
        in_specs=[
            pl.BlockSpec((BT, D), lambda i: (i, 0)),
            pl.BlockSpec((D, H), lambda i: (0, 0)),
            pl.BlockSpec((D, H), lambda i: (0, 0)),
            pl.BlockSpec((H, D), lambda i: (0, 0)),
            pl.BlockSpec((1, BT, D), lambda i: (0, i, 0)),
            pl.BlockSpec((1, BT, D), lambda i: (1, i, 0)),
            pl.BlockSpec((1, 1, BT), lambda i: (i, 0, 0)),
            pl.BlockSpec((1, 1, BT), lambda i: (i + T // BT, 0, 0)),
        ],
        out_specs=pl.BlockSpec((BT, D), lambda i: (i, 0)),
        out_shape=jax.ShapeDtypeStruct((T, D), jnp.float32),
        interpret=INTERPRET,
    )(x, sg, su, sd, y_pairs, y_pairs, w3, w3)


NW = 32
TPW = T // NW
NCH = TPW // 16


def _sc_mesh():
    return plsc.VectorSubcoreMesh(core_axis_name="c", subcore_axis_name="s")


def _dispatch(x, dest_k4):
    @functools.partial(
        pl.kernel,
        mesh=_sc_mesh(),
        out_type=jax.ShapeDtypeStruct((P_PAD, D), jnp.float32),
        scratch_types=[
            pltpu.VMEM((NCH, 16), jnp.int32),
            pltpu.VMEM((NCH, 16), jnp.int32),
            pltpu.VMEM((16, D), jnp.float32),
            pltpu.SemaphoreType.DMA,
        ],
    )
    def k(x_hbm, d_hbm, xs_hbm, idx0_v, idx1_v, rows_v, sem):
        w = lax.axis_index("s") * 2 + lax.axis_index("c")
        pltpu.sync_copy(d_hbm.at[0, w], idx0_v)
        pltpu.sync_copy(d_hbm.at[1, w], idx1_v)
        for ch in range(NCH):
            pltpu.sync_copy(x_hbm.at[pl.ds(w * TPW + ch * 16, 16)], rows_v)
            c0 = pltpu.async_copy(rows_v, xs_hbm.at[idx0_v.at[ch]], sem)
            c1 = pltpu.async_copy(rows_v, xs_hbm.at[idx1_v.at[ch]], sem)
            c0.wait()
            c1.wait()

    return k(x, dest_k4)


def _combine_gather(ys, dest_k4):
    @functools.partial(
        pl.kernel,
        mesh=_sc_mesh(),
        out_type=jax.ShapeDtypeStruct((K, T, D), jnp.float32),
        scratch_types=[
            pltpu.VMEM((NCH, 16), jnp.int32),
            pltpu.VMEM((16, D), jnp.float32),
            pltpu.SemaphoreType.DMA,
        ],
    )
    def k(ys_hbm, d_hbm, yp_hbm, idx_v, rows_v, sem):
        w = lax.axis_index("s") * 2 + lax.axis_index("c")
        for kk in range(K):
            pltpu.sync_copy(d_hbm.at[kk, w], idx_v)
            for ch in range(NCH):
                pltpu.async_copy(ys_hbm.at[idx_v.at[ch]], rows_v, sem).wait()
                pltpu.sync_copy(rows_v, yp_hbm.at[kk, pl.ds(w * TPW + ch * 16, 16)])

    return k(ys, dest_k4)


def kernel(hidden_states, shared_gate, shared_up, shared_down, Wg, Wu, Wd, router_w):
    b, l, d = hidden_states.shape
    x = hidden_states.reshape(T, D)

    d0, d1, w0, w1, meta = _route_rank(x, router_w)
    te = meta[0, :NT]
    valid = meta[1, :NT]

    dest_k4 = jnp.stack([d0.reshape(T), d1.reshape(T)]).reshape(K, NW, NCH, 16)

    xs = _dispatch(x, dest_k4)
    ys = _ffn(xs, Wg, Wu, Wd, te, valid)
    y_pairs = _combine_gather(ys, dest_k4)

    w3 = jnp.concatenate([w0, w1], axis=0)
    out = _shared_combine(x, shared_gate, shared_up, shared_down, y_pairs, w3)
    return out.reshape(b, l, d)
